# baseline (speedup 1.0000x reference)
import math

import jax
import jax.numpy as jnp
from jax import lax
from jax.experimental import pallas as pl
from jax.experimental.pallas import tpu as pltpu
from jax.experimental.pallas import tpu_sc as plsc

_D = 64
_DP = 128
_SCALE = math.sqrt(_D)
_NC, _NS = 2, 16
_NW = _NC * _NS


def _build(batch, seq):
    bpw = batch // _NW
    seq_lo = min(seq, 128)
    seq_hi = seq - seq_lo
    mesh = plsc.VectorSubcoreMesh(
        core_axis_name="c", subcore_axis_name="s",
        num_cores=_NC, num_subcores=_NS)

    def body(idx_hbm, table_hbm, out_hbm, idx0, idx1, den0, den1,
             pad0, pad1, sem0, sem1):
        wid = lax.axis_index("s") * _NC + lax.axis_index("c")
        b0 = wid * bpw

        def fire(idx_v, den_v, sem, bb):
            pltpu.sync_copy(idx_hbm.at[pl.ds(bb * seq, seq)], idx_v)
            pltpu.async_copy(table_hbm.at[idx_v.at[pl.ds(0, seq_lo)]],
                             den_v.at[pl.ds(0, seq_lo)], sem)
            if seq_hi:
                pltpu.async_copy(table_hbm.at[idx_v.at[pl.ds(seq_lo, seq_hi)]],
                                 den_v.at[pl.ds(seq_lo, seq_hi)], sem)

        fire(idx0, den0, sem0, b0)

        def step(idx_v, den_v, pad_v, sem, idx_n, den_n, sem_n, g):
            bb = b0 + g
            pltpu.make_async_copy(out_hbm.at[bb, :, pl.ds(0, _D)], den_v,
                                  sem).wait()

            @pl.when(g + 1 < bpw)
            def _():
                fire(idx_n, den_n, sem_n, bb + 1)

            def scale(s2, c):
                for u in range(2):
                    for k in range(_D // 16):
                        sl = pl.ds(k * 16, 16)
                        pad_v[s2 * 2 + u, sl] = den_v[s2 * 2 + u, sl] * _SCALE
                return c

            lax.fori_loop(0, seq // 2, scale, 0)
            pltpu.sync_copy(pad_v, out_hbm.at[bb])

        def loop(g2, carry):
            step(idx0, den0, pad0, sem0, idx1, den1, sem1, 2 * g2)
            step(idx1, den1, pad1, sem1, idx0, den0, sem0, 2 * g2 + 1)
            return carry

        lax.fori_loop(0, bpw // 2, loop, 0)

    return pl.kernel(
        body,
        out_type=jax.ShapeDtypeStruct((batch, seq, _DP), jnp.float32),
        mesh=mesh,
        compiler_params=pltpu.CompilerParams(use_tc_tiling_on_sc=False),
        scratch_types=[
            pltpu.VMEM((seq,), jnp.int32),
            pltpu.VMEM((seq,), jnp.int32),
            pltpu.VMEM((seq, _D), jnp.float32),
            pltpu.VMEM((seq, _D), jnp.float32),
            pltpu.VMEM((seq, _DP), jnp.float32),
            pltpu.VMEM((seq, _DP), jnp.float32),
            pltpu.SemaphoreType.DMA,
            pltpu.SemaphoreType.DMA,
        ],
    )


def kernel(token_ids_batch, embeddings_table):
    b, s = token_ids_batch.shape
    idx = token_ids_batch.astype(jnp.int32).reshape(b * s)
    out = _build(b, s)(idx, embeddings_table)
    return out[:, :, :_D]


# speedup vs baseline: 1.4288x; 1.4288x over previous
import math

import jax
import jax.numpy as jnp
from jax import lax
from jax.experimental import pallas as pl
from jax.experimental.pallas import tpu as pltpu
from jax.experimental.pallas import tpu_sc as plsc

_D = 64
_DP = 128
_SCALE = math.sqrt(_D)
_NC, _NS = 2, 16
_NW = _NC * _NS
_NB = 2


def _build(batch, seq):
    bpw = batch // _NW
    n_chunks = bpw // _NB
    seq_lo = min(seq, 128)
    seq_hi = seq - seq_lo
    mesh = plsc.VectorSubcoreMesh(
        core_axis_name="c", subcore_axis_name="s",
        num_cores=_NC, num_subcores=_NS)

    def body(idx_hbm, table_hbm, out_hbm, idx0, idx1, pad0, pad1,
             gsem0, gsem1, wsem0, wsem1):
        wid = lax.axis_index("s") * _NC + lax.axis_index("c")
        b0 = wid * bpw

        def fire(idx_v, pad_v, gsem, bb):
            pltpu.sync_copy(idx_hbm.at[pl.ds(bb * seq, _NB * seq)], idx_v)
            for r in range(_NB):
                pltpu.async_copy(
                    table_hbm.at[idx_v.at[pl.ds(r * seq, seq_lo)]],
                    pad_v.at[r, pl.ds(0, seq_lo)], gsem)
                if seq_hi:
                    pltpu.async_copy(
                        table_hbm.at[idx_v.at[pl.ds(r * seq + seq_lo,
                                                    seq_hi)]],
                        pad_v.at[r, pl.ds(seq_lo, seq_hi)], gsem)

        fire(idx0, pad0, gsem0, b0)

        def step(idx_v, pad_v, gsem, wsem, idx_n, pad_n, gsem_n, wsem_n, g):
            bb = b0 + g * _NB
            pltpu.make_async_copy(out_hbm.at[pl.ds(bb, _NB)], pad_v,
                                  gsem).wait()

            @pl.when(g + 1 < n_chunks)
            def _():
                @pl.when(g >= 1)
                def _():
                    pltpu.make_async_copy(
                        pad_n, out_hbm.at[pl.ds(bb - _NB, _NB)],
                        wsem_n).wait()
                fire(idx_n, pad_n, gsem_n, bb + _NB)

            def scale(s2, c):
                for u in range(2):
                    for r in range(_NB):
                        for k in range(_D // 16):
                            sl = pl.ds(k * 16, 16)
                            pad_v[r, s2 * 2 + u, sl] = (
                                pad_v[r, s2 * 2 + u, sl] * _SCALE)
                return c

            lax.fori_loop(0, seq // 2, scale, 0)
            pltpu.async_copy(pad_v, out_hbm.at[pl.ds(bb, _NB)], wsem)

        def loop(g2, carry):
            step(idx0, pad0, gsem0, wsem0, idx1, pad1, gsem1, wsem1, 2 * g2)
            step(idx1, pad1, gsem1, wsem1, idx0, pad0, gsem0, wsem0,
                 2 * g2 + 1)
            return carry

        lax.fori_loop(0, n_chunks // 2, loop, 0)
        last = b0 + (n_chunks - 2) * _NB
        pltpu.make_async_copy(pad0, out_hbm.at[pl.ds(last, _NB)],
                              wsem0).wait()
        pltpu.make_async_copy(pad1, out_hbm.at[pl.ds(last + _NB, _NB)],
                              wsem1).wait()

    return pl.kernel(
        body,
        out_type=jax.ShapeDtypeStruct((batch, seq, _DP), jnp.float32),
        mesh=mesh,
        compiler_params=pltpu.CompilerParams(use_tc_tiling_on_sc=False),
        scratch_types=[
            pltpu.VMEM((_NB * seq,), jnp.int32),
            pltpu.VMEM((_NB * seq,), jnp.int32),
            pltpu.VMEM((_NB, seq, _DP), jnp.float32),
            pltpu.VMEM((_NB, seq, _DP), jnp.float32),
            pltpu.SemaphoreType.DMA,
            pltpu.SemaphoreType.DMA,
            pltpu.SemaphoreType.DMA,
            pltpu.SemaphoreType.DMA,
        ],
    )


def kernel(token_ids_batch, embeddings_table):
    b, s = token_ids_batch.shape
    v = embeddings_table.shape[0]
    idx = token_ids_batch.astype(jnp.int32).reshape(b * s)
    table_p = jnp.pad(embeddings_table, ((0, 0), (0, _DP - _D)))
    out = _build(b, s)(idx, table_p, )
    return out[:, :, :_D]


# speedup vs baseline: 1.4517x; 1.0160x over previous
import math

import jax
import jax.numpy as jnp
from jax import lax
from jax.experimental import pallas as pl
from jax.experimental.pallas import tpu as pltpu
from jax.experimental.pallas import tpu_sc as plsc

_D = 64
_DP = 128
_SCALE = math.sqrt(_D)
_NC, _NS = 2, 16
_NW = _NC * _NS
_NR = 4


def _build(batch, seq):
    bpw = batch // _NW
    seq_lo = min(seq, 128)
    seq_hi = seq - seq_lo
    mesh = plsc.VectorSubcoreMesh(
        core_axis_name="c", subcore_axis_name="s",
        num_cores=_NC, num_subcores=_NS)

    def body(idx_hbm, table_hbm, out_hbm, *refs):
        idx_v = refs[0:_NR]
        pad_v = refs[_NR:2 * _NR]
        gsem = refs[2 * _NR:3 * _NR]
        wsem = refs[3 * _NR:4 * _NR]
        wid = lax.axis_index("s") * _NC + lax.axis_index("c")
        b0 = wid * bpw

        def fire(j, bb):
            pltpu.sync_copy(idx_hbm.at[pl.ds(bb * seq, seq)], idx_v[j])
            pltpu.async_copy(
                table_hbm.at[idx_v[j].at[pl.ds(0, seq_lo)]],
                pad_v[j].at[pl.ds(0, seq_lo)], gsem[j])
            if seq_hi:
                pltpu.async_copy(
                    table_hbm.at[idx_v[j].at[pl.ds(seq_lo, seq_hi)]],
                    pad_v[j].at[pl.ds(seq_lo, seq_hi)], gsem[j])

        for j in range(2):
            fire(j, b0 + j)

        def step(j, g):
            bb = b0 + g
            nxt = (j + 2) % _NR

            @pl.when(g + 2 < bpw)
            def _():
                @pl.when(g >= 2)
                def _():
                    pltpu.make_async_copy(
                        pad_v[nxt], out_hbm.at[bb - 2], wsem[nxt]).wait()
                fire(nxt, bb + 2)

            pltpu.make_async_copy(out_hbm.at[bb], pad_v[j], gsem[j]).wait()

            def scale(s4, c):
                for u in range(4):
                    for k in range(_D // 16):
                        sl = pl.ds(k * 16, 16)
                        pad_v[j][s4 * 4 + u, sl] = (
                            pad_v[j][s4 * 4 + u, sl] * _SCALE)
                return c

            lax.fori_loop(0, seq // 4, scale, 0)
            pltpu.async_copy(pad_v[j], out_hbm.at[bb], wsem[j])

        def loop(g4, carry):
            for j in range(_NR):
                step(j, g4 * _NR + j)
            return carry

        lax.fori_loop(0, bpw // _NR, loop, 0)
        for j in range(_NR):
            pltpu.make_async_copy(pad_v[j], out_hbm.at[b0 + bpw - _NR + j],
                                  wsem[j]).wait()

    return pl.kernel(
        body,
        out_type=jax.ShapeDtypeStruct((batch, seq, _DP), jnp.float32),
        mesh=mesh,
        compiler_params=pltpu.CompilerParams(use_tc_tiling_on_sc=False),
        scratch_types=(
            [pltpu.VMEM((seq,), jnp.int32) for _ in range(_NR)]
            + [pltpu.VMEM((seq, _DP), jnp.float32) for _ in range(_NR)]
            + [pltpu.SemaphoreType.DMA for _ in range(2 * _NR)]
        ),
    )


def kernel(token_ids_batch, embeddings_table):
    b, s = token_ids_batch.shape
    idx = token_ids_batch.astype(jnp.int32).reshape(b * s)
    table_p = jnp.pad(embeddings_table, ((0, 0), (0, _DP - _D)))
    out = _build(b, s)(idx, table_p)
    return out[:, :, :_D]
